# baseline (speedup 1.0000x reference)
import functools

import jax
import jax.numpy as jnp
from jax import lax
from jax.experimental import pallas as pl
from jax.experimental.pallas import tpu as pltpu
from jax.experimental.pallas import tpu_sc as plsc

N_NODES = 10000
N_EDGES = 320000
D = 128

NC = 2
NS = 16
NW = NC * NS

N_PAD = 10240
ROWS_PER_TILE = N_PAD // NS
K = 128
E_PAD = 327680
EC = E_PAD // NW // K
CB = EC // 2
B_PAD = 10240
BW = B_PAD // NW
KB = 80
BQ = BW // KB

_mesh = plsc.VectorSubcoreMesh(core_axis_name="c", subcore_axis_name="s")


@functools.partial(
    pl.kernel,
    out_type=jax.ShapeDtypeStruct((NC, N_PAD, D), jnp.float32),
    mesh=_mesh,
    scratch_types=[
        pltpu.VMEM((CB, K), jnp.int32),
        pltpu.VMEM((CB, K), jnp.int32),
        pltpu.VMEM((K, D), jnp.float32),
        pltpu.VMEM((K, D), jnp.float32),
        pltpu.VMEM_SHARED((N_PAD, D), jnp.float32),
        pltpu.SemaphoreType.DMA,
    ],
)
def _edge_agg(emb_hbm, src_hbm, dst_hbm, acc_out,
              src_v, dst_v, rows0_v, rows1_v, acc_sh, sem0):
    c = lax.axis_index("c")
    s = lax.axis_index("s")
    zero16 = jnp.zeros((16,), jnp.float32)

    def _zero_rows(i, _):
        for q in range(D // 16):
            rows0_v[i, pl.ds(q * 16, 16)] = zero16
        return 0

    lax.fori_loop(0, K, _zero_rows, 0)
    my_base = s * ROWS_PER_TILE
    for q in range(ROWS_PER_TILE // K):
        pltpu.sync_copy(rows0_v, acc_sh.at[pl.ds(my_base + q * K, K)])
    plsc.subcore_barrier()

    for h in range(EC // CB):
        pltpu.sync_copy(src_hbm.at[c, s, pl.ds(h * CB, CB)], src_v)
        pltpu.sync_copy(dst_hbm.at[c, s, pl.ds(h * CB, CB)], dst_v)

        def _chunk(j, _):
            pltpu.sync_copy(emb_hbm.at[src_v.at[j]], rows0_v)
            pltpu.sync_copy(rows0_v, acc_sh.at[dst_v.at[j]], add=True)
            return 0

        lax.fori_loop(0, CB, _chunk, 0)
    plsc.subcore_barrier()

    pltpu.sync_copy(acc_sh.at[pl.ds(my_base, ROWS_PER_TILE)],
                    acc_out.at[c, pl.ds(my_base, ROWS_PER_TILE)])


@functools.partial(
    pl.kernel,
    out_type=jax.ShapeDtypeStruct((NC, N_PAD, D), jnp.float32),
    mesh=_mesh,
    scratch_types=[
        pltpu.VMEM((EC, K), jnp.int32),
        pltpu.VMEM((K, D), jnp.float32),
        pltpu.VMEM_SHARED((N_PAD, D), jnp.float32),
    ],
)
def _degrees(dst_hbm, deg_out, dst_v, ones_v, deg_sh):
    c = lax.axis_index("c")
    s = lax.axis_index("s")
    zero16 = jnp.zeros((16,), jnp.float32)

    def _fill(val):
        def _f(i, _):
            for q in range(D // 16):
                ones_v[i, pl.ds(q * 16, 16)] = zero16 + val
            return 0
        lax.fori_loop(0, K, _f, 0)

    _fill(0.0)
    my_base = s * ROWS_PER_TILE
    for q in range(ROWS_PER_TILE // K):
        pltpu.sync_copy(ones_v, deg_sh.at[pl.ds(my_base + q * K, K)])
    _fill(1.0)
    plsc.subcore_barrier()

    pltpu.sync_copy(dst_hbm.at[c, s], dst_v)

    def _chunk(j, _):
        pltpu.sync_copy(ones_v, deg_sh.at[dst_v.at[j]], add=True)
        return 0

    lax.fori_loop(0, EC, _chunk, 0)
    plsc.subcore_barrier()

    pltpu.sync_copy(deg_sh.at[pl.ds(my_base, ROWS_PER_TILE)],
                    deg_out.at[c, pl.ds(my_base, ROWS_PER_TILE)])


def _linear_body(emb_ref, acc_ref, deg_ref, wt_ref, out_ref):
    a = acc_ref[0] + acc_ref[1]
    dsum = deg_ref[0] + deg_ref[1]
    deg = jnp.max(dsum, axis=1, keepdims=True)
    r = 1.0 / jnp.maximum(deg, 1.0)
    x1 = emb_ref[...]
    x2 = a * r
    w1 = wt_ref[:D, :]
    w2 = wt_ref[D:, :]
    t = (jnp.dot(x1, w1, preferred_element_type=jnp.float32,
                 precision=lax.Precision.HIGHEST)
         + jnp.dot(x2, w2, preferred_element_type=jnp.float32,
                   precision=lax.Precision.HIGHEST))
    out_ref[...] = jnp.maximum(t, 0.0)


@functools.partial(
    pl.kernel,
    out_type=jax.ShapeDtypeStruct((B_PAD, D), jnp.float32),
    mesh=_mesh,
    scratch_types=[
        pltpu.VMEM((BQ, KB), jnp.int32),
        pltpu.VMEM((BW, D), jnp.float32),
    ],
)
def _batch_gather(t_hbm, nid_hbm, out_hbm, idx_v, rows_v):
    c = lax.axis_index("c")
    s = lax.axis_index("s")
    wid = c * NS + s
    pltpu.sync_copy(nid_hbm.at[c, s], idx_v)
    for q in range(BQ):
        pltpu.sync_copy(t_hbm.at[idx_v.at[q]], rows_v.at[pl.ds(q * KB, KB)])
    pltpu.sync_copy(rows_v, out_hbm.at[pl.ds(wid * BW, BW)])


def kernel(node_ids, edge_index, emb_table, W):
    npad = E_PAD - N_EDGES
    src = jnp.concatenate(
        [edge_index[0], jnp.zeros((npad,), edge_index.dtype)]
    ).reshape(NC, NS, EC, K)
    dst = jnp.concatenate(
        [edge_index[1], jnp.full((npad,), N_NODES, edge_index.dtype)]
    ).reshape(NC, NS, EC, K)
    acc = _edge_agg(emb_table, src, dst)
    deg = _degrees(dst)

    R = 1000
    t = pl.pallas_call(
        _linear_body,
        grid=(N_NODES // R,),
        in_specs=[
            pl.BlockSpec((R, D), lambda i: (i, 0)),
            pl.BlockSpec((NC, R, D), lambda i: (0, i, 0)),
            pl.BlockSpec((NC, R, D), lambda i: (0, i, 0)),
            pl.BlockSpec((2 * D, D), lambda i: (0, 0)),
        ],
        out_specs=pl.BlockSpec((R, D), lambda i: (i, 0)),
        out_shape=jax.ShapeDtypeStruct((N_NODES, D), jnp.float32),
    )(emb_table, acc, deg, W.T)

    nid = jnp.concatenate(
        [node_ids, jnp.zeros((B_PAD - N_NODES,), node_ids.dtype)]
    ).reshape(NC, NS, BQ, KB)
    out = _batch_gather(t, nid)
    return out[:N_NODES]


# speedup vs baseline: 1.0189x; 1.0189x over previous
import functools

import jax
import jax.numpy as jnp
from jax import lax
from jax.experimental import pallas as pl
from jax.experimental.pallas import tpu as pltpu
from jax.experimental.pallas import tpu_sc as plsc

N_NODES = 10000
N_EDGES = 320000
D = 128

NC = 2
NS = 16
NW = NC * NS

N_PAD = 10240
ROWS_PER_TILE = N_PAD // NS
K = 80
E_PAD = 327680
EC = E_PAD // NW // K
CB = EC // 4
B_PAD = 10240
BW = B_PAD // NW
KB = 80
BQ = BW // KB

_mesh = plsc.VectorSubcoreMesh(core_axis_name="c", subcore_axis_name="s")


@functools.partial(
    pl.kernel,
    out_type=jax.ShapeDtypeStruct((NC, N_PAD, D), jnp.float32),
    mesh=_mesh,
    scratch_types=[
        pltpu.VMEM((CB, K), jnp.int32),
        pltpu.VMEM((CB, K), jnp.int32),
        pltpu.VMEM((K, D), jnp.float32),
        pltpu.VMEM((K, D), jnp.float32),
        pltpu.VMEM_SHARED((N_PAD, D), jnp.float32),
        pltpu.SemaphoreType.DMA,
    ],
)
def _edge_agg(emb_hbm, src_hbm, dst_hbm, acc_out,
              src_v, dst_v, rows0_v, rows1_v, acc_sh, sem0):
    c = lax.axis_index("c")
    s = lax.axis_index("s")
    zero16 = jnp.zeros((16,), jnp.float32)

    def _zero_rows(i, _):
        for q in range(D // 16):
            rows0_v[i, pl.ds(q * 16, 16)] = zero16
        return 0

    lax.fori_loop(0, K, _zero_rows, 0)
    my_base = s * ROWS_PER_TILE
    for q in range(ROWS_PER_TILE // K):
        pltpu.sync_copy(rows0_v, acc_sh.at[pl.ds(my_base + q * K, K)])
    plsc.subcore_barrier()

    for h in range(EC // CB):
        pltpu.sync_copy(src_hbm.at[c, s, pl.ds(h * CB, CB)], src_v)
        pltpu.sync_copy(dst_hbm.at[c, s, pl.ds(h * CB, CB)], dst_v)
        pltpu.async_copy(emb_hbm.at[src_v.at[0]], rows0_v, sem0)
        last = CB - 1

        def _pair(jj, _):
            j = 2 * jj
            pltpu.make_async_copy(emb_hbm.at[src_v.at[0]], rows0_v, sem0).wait()
            pltpu.async_copy(emb_hbm.at[src_v.at[j + 1]], rows1_v, sem0)
            pltpu.sync_copy(rows0_v, acc_sh.at[dst_v.at[j]], add=True)
            pltpu.make_async_copy(emb_hbm.at[src_v.at[0]], rows1_v, sem0).wait()
            pltpu.async_copy(emb_hbm.at[src_v.at[jnp.minimum(j + 2, last)]],
                             rows0_v, sem0)
            pltpu.sync_copy(rows1_v, acc_sh.at[dst_v.at[j + 1]], add=True)
            return 0

        lax.fori_loop(0, CB // 2, _pair, 0)
        pltpu.make_async_copy(emb_hbm.at[src_v.at[0]], rows0_v, sem0).wait()
    plsc.subcore_barrier()

    pltpu.sync_copy(acc_sh.at[pl.ds(my_base, ROWS_PER_TILE)],
                    acc_out.at[c, pl.ds(my_base, ROWS_PER_TILE)])


@functools.partial(
    pl.kernel,
    out_type=jax.ShapeDtypeStruct((NC, N_PAD, D), jnp.float32),
    mesh=_mesh,
    scratch_types=[
        pltpu.VMEM((EC, K), jnp.int32),
        pltpu.VMEM((K, D), jnp.float32),
        pltpu.VMEM_SHARED((N_PAD, D), jnp.float32),
    ],
)
def _degrees(dst_hbm, deg_out, dst_v, ones_v, deg_sh):
    c = lax.axis_index("c")
    s = lax.axis_index("s")
    zero16 = jnp.zeros((16,), jnp.float32)

    def _fill(val):
        def _f(i, _):
            for q in range(D // 16):
                ones_v[i, pl.ds(q * 16, 16)] = zero16 + val
            return 0
        lax.fori_loop(0, K, _f, 0)

    _fill(0.0)
    my_base = s * ROWS_PER_TILE
    for q in range(ROWS_PER_TILE // K):
        pltpu.sync_copy(ones_v, deg_sh.at[pl.ds(my_base + q * K, K)])
    _fill(1.0)
    plsc.subcore_barrier()

    pltpu.sync_copy(dst_hbm.at[c, s], dst_v)

    def _chunk(j, _):
        pltpu.sync_copy(ones_v, deg_sh.at[dst_v.at[j]], add=True)
        return 0

    lax.fori_loop(0, EC, _chunk, 0)
    plsc.subcore_barrier()

    pltpu.sync_copy(deg_sh.at[pl.ds(my_base, ROWS_PER_TILE)],
                    deg_out.at[c, pl.ds(my_base, ROWS_PER_TILE)])


def _linear_body(emb_ref, acc_ref, deg_ref, wt_ref, out_ref):
    a = acc_ref[0] + acc_ref[1]
    dsum = deg_ref[0] + deg_ref[1]
    deg = jnp.max(dsum, axis=1, keepdims=True)
    r = 1.0 / jnp.maximum(deg, 1.0)
    x1 = emb_ref[...]
    x2 = a * r
    w1 = wt_ref[:D, :]
    w2 = wt_ref[D:, :]
    t = (jnp.dot(x1, w1, preferred_element_type=jnp.float32,
                 precision=lax.Precision.HIGHEST)
         + jnp.dot(x2, w2, preferred_element_type=jnp.float32,
                   precision=lax.Precision.HIGHEST))
    out_ref[...] = jnp.maximum(t, 0.0)


@functools.partial(
    pl.kernel,
    out_type=jax.ShapeDtypeStruct((B_PAD, D), jnp.float32),
    mesh=_mesh,
    scratch_types=[
        pltpu.VMEM((BQ, KB), jnp.int32),
        pltpu.VMEM((BW, D), jnp.float32),
    ],
)
def _batch_gather(t_hbm, nid_hbm, out_hbm, idx_v, rows_v):
    c = lax.axis_index("c")
    s = lax.axis_index("s")
    wid = c * NS + s
    pltpu.sync_copy(nid_hbm.at[c, s], idx_v)
    for q in range(BQ):
        pltpu.sync_copy(t_hbm.at[idx_v.at[q]], rows_v.at[pl.ds(q * KB, KB)])
    pltpu.sync_copy(rows_v, out_hbm.at[pl.ds(wid * BW, BW)])


def kernel(node_ids, edge_index, emb_table, W):
    npad = E_PAD - N_EDGES
    src = jnp.concatenate(
        [edge_index[0], jnp.zeros((npad,), edge_index.dtype)]
    ).reshape(NC, NS, EC, K)
    dst = jnp.concatenate(
        [edge_index[1], jnp.full((npad,), N_NODES, edge_index.dtype)]
    ).reshape(NC, NS, EC, K)
    acc = _edge_agg(emb_table, src, dst)
    deg = _degrees(dst)

    R = 1000
    t = pl.pallas_call(
        _linear_body,
        grid=(N_NODES // R,),
        in_specs=[
            pl.BlockSpec((R, D), lambda i: (i, 0)),
            pl.BlockSpec((NC, R, D), lambda i: (0, i, 0)),
            pl.BlockSpec((NC, R, D), lambda i: (0, i, 0)),
            pl.BlockSpec((2 * D, D), lambda i: (0, 0)),
        ],
        out_specs=pl.BlockSpec((R, D), lambda i: (i, 0)),
        out_shape=jax.ShapeDtypeStruct((N_NODES, D), jnp.float32),
    )(emb_table, acc, deg, W.T)

    nid = jnp.concatenate(
        [node_ids, jnp.zeros((B_PAD - N_NODES,), node_ids.dtype)]
    ).reshape(NC, NS, BQ, KB)
    out = _batch_gather(t, nid)
    return out[:N_NODES]


# speedup vs baseline: 1.0198x; 1.0010x over previous
import functools

import jax
import jax.numpy as jnp
from jax import lax
from jax.experimental import pallas as pl
from jax.experimental.pallas import tpu as pltpu
from jax.experimental.pallas import tpu_sc as plsc

N_NODES = 10000
N_EDGES = 320000
D = 128

NC = 2
NS = 16
NW = NC * NS

N_PAD = 10240
ROWS_PER_TILE = N_PAD // NS
K = 80
E_PAD = 327680
EC = E_PAD // NW // K
CB = EC // 4
B_PAD = 10240
BW = B_PAD // NW
KB = 80
BQ = BW // KB

_mesh = plsc.VectorSubcoreMesh(core_axis_name="c", subcore_axis_name="s")


@functools.partial(
    pl.kernel,
    out_type=jax.ShapeDtypeStruct((NC, N_PAD, D), jnp.float32),
    mesh=_mesh,
    scratch_types=[
        pltpu.VMEM((CB, K), jnp.int32),
        pltpu.VMEM((CB, K), jnp.int32),
        pltpu.VMEM((K, D), jnp.float32),
        pltpu.VMEM((K, D), jnp.float32),
        pltpu.VMEM_SHARED((N_PAD, D), jnp.float32),
        pltpu.SemaphoreType.DMA,
    ],
)
def _edge_agg(emb_hbm, src_hbm, dst_hbm, acc_out,
              src_v, dst_v, rows0_v, rows1_v, acc_sh, sem0):
    c = lax.axis_index("c")
    s = lax.axis_index("s")
    zero16 = jnp.zeros((16,), jnp.float32)

    def _zero_rows(i, _):
        for q in range(D // 16):
            rows0_v[i, pl.ds(q * 16, 16)] = zero16
        return 0

    lax.fori_loop(0, K, _zero_rows, 0)
    my_base = s * ROWS_PER_TILE
    for q in range(ROWS_PER_TILE // K):
        pltpu.sync_copy(rows0_v, acc_sh.at[pl.ds(my_base + q * K, K)])
    plsc.subcore_barrier()

    for h in range(EC // CB):
        pltpu.sync_copy(src_hbm.at[c, s, pl.ds(h * CB, CB)], src_v)
        pltpu.sync_copy(dst_hbm.at[c, s, pl.ds(h * CB, CB)], dst_v)
        pltpu.async_copy(emb_hbm.at[src_v.at[0]], rows0_v, sem0)
        last = CB - 1

        def _pair(jj, _):
            j = 2 * jj
            pltpu.make_async_copy(emb_hbm.at[src_v.at[0]], rows0_v, sem0).wait()
            pltpu.async_copy(emb_hbm.at[src_v.at[j + 1]], rows1_v, sem0)
            pltpu.sync_copy(rows0_v, acc_sh.at[dst_v.at[j]], add=True)
            pltpu.make_async_copy(emb_hbm.at[src_v.at[0]], rows1_v, sem0).wait()
            pltpu.async_copy(emb_hbm.at[src_v.at[jnp.minimum(j + 2, last)]],
                             rows0_v, sem0)
            pltpu.sync_copy(rows1_v, acc_sh.at[dst_v.at[j + 1]], add=True)
            return 0

        lax.fori_loop(0, CB // 2, _pair, 0)
        pltpu.make_async_copy(emb_hbm.at[src_v.at[0]], rows0_v, sem0).wait()
    plsc.subcore_barrier()

    pltpu.sync_copy(acc_sh.at[pl.ds(my_base, ROWS_PER_TILE)],
                    acc_out.at[c, pl.ds(my_base, ROWS_PER_TILE)])


@functools.partial(
    pl.kernel,
    out_type=jax.ShapeDtypeStruct((NC, N_PAD, D), jnp.float32),
    mesh=_mesh,
    scratch_types=[
        pltpu.VMEM((EC, K), jnp.int32),
        pltpu.VMEM((K, D), jnp.float32),
        pltpu.VMEM_SHARED((N_PAD, D), jnp.float32),
    ],
)
def _degrees(dst_hbm, deg_out, dst_v, ones_v, deg_sh):
    c = lax.axis_index("c")
    s = lax.axis_index("s")
    zero16 = jnp.zeros((16,), jnp.float32)

    def _fill(val):
        def _f(i, _):
            for q in range(D // 16):
                ones_v[i, pl.ds(q * 16, 16)] = zero16 + val
            return 0
        lax.fori_loop(0, K, _f, 0)

    _fill(0.0)
    my_base = s * ROWS_PER_TILE
    for q in range(ROWS_PER_TILE // K):
        pltpu.sync_copy(ones_v, deg_sh.at[pl.ds(my_base + q * K, K)])
    _fill(1.0)
    plsc.subcore_barrier()

    pltpu.sync_copy(dst_hbm.at[c, s], dst_v)

    def _chunk(j, _):
        pltpu.sync_copy(ones_v, deg_sh.at[dst_v.at[j]], add=True)
        return 0

    lax.fori_loop(0, EC, _chunk, 0)
    plsc.subcore_barrier()

    pltpu.sync_copy(deg_sh.at[pl.ds(my_base, ROWS_PER_TILE)],
                    deg_out.at[c, pl.ds(my_base, ROWS_PER_TILE)])


def _linear_body(emb_ref, acc_ref, deg_ref, wt_ref, out_ref):
    a = acc_ref[0] + acc_ref[1]
    dsum = deg_ref[0] + deg_ref[1]
    deg = jnp.max(dsum, axis=1, keepdims=True)
    r = 1.0 / jnp.maximum(deg, 1.0)
    x1 = emb_ref[...]
    x2 = a * r
    w1 = wt_ref[:D, :]
    w2 = wt_ref[D:, :]
    t = (jnp.dot(x1, w1, preferred_element_type=jnp.float32,
                 precision=lax.Precision.HIGHEST)
         + jnp.dot(x2, w2, preferred_element_type=jnp.float32,
                   precision=lax.Precision.HIGHEST))
    out_ref[...] = jnp.maximum(t, 0.0)


@functools.partial(
    pl.kernel,
    out_type=jax.ShapeDtypeStruct((B_PAD, D), jnp.float32),
    mesh=_mesh,
    scratch_types=[
        pltpu.VMEM((BQ, KB), jnp.int32),
        pltpu.VMEM((BW, D), jnp.float32),
    ],
)
def _batch_gather(t_hbm, nid_hbm, out_hbm, idx_v, rows_v):
    c = lax.axis_index("c")
    s = lax.axis_index("s")
    wid = c * NS + s
    pltpu.sync_copy(nid_hbm.at[c, s], idx_v)
    for q in range(BQ):
        pltpu.sync_copy(t_hbm.at[idx_v.at[q]], rows_v.at[pl.ds(q * KB, KB)])
    pltpu.sync_copy(rows_v, out_hbm.at[pl.ds(wid * BW, BW)])


def kernel(node_ids, edge_index, emb_table, W):
    npad = E_PAD - N_EDGES
    src = jnp.concatenate(
        [edge_index[0], jnp.zeros((npad,), edge_index.dtype)]
    ).reshape(NC, NS, EC, K)
    pad_dst = N_NODES + (jnp.arange(npad, dtype=edge_index.dtype)
                         % (N_PAD - N_NODES))
    dst = jnp.concatenate([edge_index[1], pad_dst]).reshape(NC, NS, EC, K)
    acc = _edge_agg(emb_table, src, dst)
    deg = _degrees(dst)

    R = 1000
    t = pl.pallas_call(
        _linear_body,
        grid=(N_NODES // R,),
        in_specs=[
            pl.BlockSpec((R, D), lambda i: (i, 0)),
            pl.BlockSpec((NC, R, D), lambda i: (0, i, 0)),
            pl.BlockSpec((NC, R, D), lambda i: (0, i, 0)),
            pl.BlockSpec((2 * D, D), lambda i: (0, 0)),
        ],
        out_specs=pl.BlockSpec((R, D), lambda i: (i, 0)),
        out_shape=jax.ShapeDtypeStruct((N_NODES, D), jnp.float32),
    )(emb_table, acc, deg, W.T)

    nid = jnp.concatenate(
        [node_ids, jnp.zeros((B_PAD - N_NODES,), node_ids.dtype)]
    ).reshape(NC, NS, BQ, KB)
    out = _batch_gather(t, nid)
    return out[:N_NODES]


# speedup vs baseline: 1.6282x; 1.5965x over previous
import functools

import jax
import jax.numpy as jnp
from jax import lax
from jax.experimental import pallas as pl
from jax.experimental.pallas import tpu as pltpu
from jax.experimental.pallas import tpu_sc as plsc

N_NODES = 10000
N_EDGES = 320000
D = 128

NC = 2
NS = 16
NW = NC * NS

N_PAD = 10240
ROWS_PER_TILE = N_PAD // NS
K = 80
E_PAD = 327680
EC = E_PAD // NW // K
B_PAD = 10240
BW = B_PAD // NW
BQ = BW // K

_mesh = plsc.VectorSubcoreMesh(core_axis_name="c", subcore_axis_name="s")


@functools.partial(
    pl.kernel,
    out_type=jax.ShapeDtypeStruct((NC, N_PAD, D), jnp.float32),
    mesh=_mesh,
    scratch_types=[
        pltpu.VMEM((EC, K), jnp.int32),
        pltpu.VMEM((EC, K), jnp.int32),
        pltpu.VMEM((K, D), jnp.float32),
        pltpu.VMEM_SHARED((N_PAD, D), jnp.float32),
    ],
)
def _edge_agg(emb_hbm, src_hbm, dst_hbm, acc_out, src_v, dst_v, rows_v, acc_sh):
    c = lax.axis_index("c")
    s = lax.axis_index("s")
    zero16 = jnp.zeros((16,), jnp.float32)

    def _zero_rows(i, _):
        for q in range(D // 16):
            rows_v[i, pl.ds(q * 16, 16)] = zero16
        return 0

    lax.fori_loop(0, K, _zero_rows, 0)
    my_base = s * ROWS_PER_TILE
    for q in range(ROWS_PER_TILE // K):
        pltpu.sync_copy(rows_v, acc_sh.at[pl.ds(my_base + q * K, K)])
    plsc.subcore_barrier()

    pltpu.sync_copy(src_hbm.at[c, s], src_v)
    pltpu.sync_copy(dst_hbm.at[c, s], dst_v)

    def _chunk(j, _):
        pltpu.sync_copy(emb_hbm.at[src_v.at[j]], rows_v)
        pltpu.sync_copy(rows_v, acc_sh.at[dst_v.at[j]], add=True)
        return 0

    lax.fori_loop(0, EC, _chunk, 0)
    plsc.subcore_barrier()

    pltpu.sync_copy(acc_sh.at[pl.ds(my_base, ROWS_PER_TILE)],
                    acc_out.at[c, pl.ds(my_base, ROWS_PER_TILE)])


@functools.partial(
    pl.kernel,
    out_type=jax.ShapeDtypeStruct((NC, N_PAD, D), jnp.float32),
    mesh=_mesh,
    scratch_types=[
        pltpu.VMEM((EC, K), jnp.int32),
        pltpu.VMEM((K, D), jnp.float32),
        pltpu.VMEM_SHARED((N_PAD, D), jnp.float32),
    ],
)
def _degrees(dst_hbm, deg_out, dst_v, ones_v, deg_sh):
    c = lax.axis_index("c")
    s = lax.axis_index("s")
    zero16 = jnp.zeros((16,), jnp.float32)

    def _fill(val):
        def _f(i, _):
            for q in range(D // 16):
                ones_v[i, pl.ds(q * 16, 16)] = zero16 + val
            return 0
        lax.fori_loop(0, K, _f, 0)

    _fill(0.0)
    my_base = s * ROWS_PER_TILE
    for q in range(ROWS_PER_TILE // K):
        pltpu.sync_copy(ones_v, deg_sh.at[pl.ds(my_base + q * K, K)])
    _fill(1.0)
    plsc.subcore_barrier()

    pltpu.sync_copy(dst_hbm.at[c, s], dst_v)

    def _chunk(j, _):
        pltpu.sync_copy(ones_v, deg_sh.at[dst_v.at[j]], add=True)
        return 0

    lax.fori_loop(0, EC, _chunk, 0)
    plsc.subcore_barrier()

    pltpu.sync_copy(deg_sh.at[pl.ds(my_base, ROWS_PER_TILE)],
                    deg_out.at[c, pl.ds(my_base, ROWS_PER_TILE)])


def _linear_body(emb_ref, acc_ref, deg_ref, wt_ref, out_ref):
    a = acc_ref[0] + acc_ref[1]
    dsum = deg_ref[0] + deg_ref[1]
    deg = jnp.max(dsum, axis=1, keepdims=True)
    r = 1.0 / jnp.maximum(deg, 1.0)
    x1 = emb_ref[...]
    x2 = a * r
    w1 = wt_ref[:D, :]
    w2 = wt_ref[D:, :]
    t = (jnp.dot(x1, w1, preferred_element_type=jnp.float32,
                 precision=lax.Precision.HIGHEST)
         + jnp.dot(x2, w2, preferred_element_type=jnp.float32,
                   precision=lax.Precision.HIGHEST))
    out_ref[...] = jnp.maximum(t, 0.0)


@functools.partial(
    pl.kernel,
    out_type=jax.ShapeDtypeStruct((B_PAD, D), jnp.float32),
    mesh=_mesh,
    scratch_types=[
        pltpu.VMEM((BQ, K), jnp.int32),
        pltpu.VMEM((BW, D), jnp.float32),
    ],
)
def _batch_gather(t_hbm, nid_hbm, out_hbm, idx_v, rows_v):
    c = lax.axis_index("c")
    s = lax.axis_index("s")
    wid = c * NS + s
    pltpu.sync_copy(nid_hbm.at[c, s], idx_v)
    for q in range(BQ):
        pltpu.sync_copy(t_hbm.at[idx_v.at[q]], rows_v.at[pl.ds(q * K, K)])
    pltpu.sync_copy(rows_v, out_hbm.at[pl.ds(wid * BW, BW)])


def kernel(node_ids, edge_index, emb_table, W):
    npad = E_PAD - N_EDGES
    ar = jnp.arange(npad, dtype=edge_index.dtype)
    src = jnp.concatenate(
        [edge_index[0], ar % N_NODES]).reshape(NC, NS, EC, K)
    dst = jnp.concatenate(
        [edge_index[1], N_NODES + ar % (N_PAD - N_NODES)]
    ).reshape(NC, NS, EC, K)
    acc = _edge_agg(emb_table, src, dst)
    deg = _degrees(dst)

    R = 1000
    t = pl.pallas_call(
        _linear_body,
        grid=(N_NODES // R,),
        in_specs=[
            pl.BlockSpec((R, D), lambda i: (i, 0)),
            pl.BlockSpec((NC, R, D), lambda i: (0, i, 0)),
            pl.BlockSpec((NC, R, D), lambda i: (0, i, 0)),
            pl.BlockSpec((2 * D, D), lambda i: (0, 0)),
        ],
        out_specs=pl.BlockSpec((R, D), lambda i: (i, 0)),
        out_shape=jax.ShapeDtypeStruct((N_NODES, D), jnp.float32),
    )(emb_table, acc, deg, W.T)

    nid = jnp.concatenate(
        [node_ids, jnp.zeros((B_PAD - N_NODES,), node_ids.dtype)]
    ).reshape(NC, NS, BQ, K)
    out = _batch_gather(t, nid)
    return out[:N_NODES]


# speedup vs baseline: 1.8838x; 1.1570x over previous
import functools

import jax
import jax.numpy as jnp
from jax import lax
from jax.experimental import pallas as pl
from jax.experimental.pallas import tpu as pltpu
from jax.experimental.pallas import tpu_sc as plsc

N_NODES = 10000
N_EDGES = 320000
D = 128

NC = 2
NS = 16
NW = NC * NS

N_PAD = 10240
ROWS_PER_TILE = N_PAD // NS
K = 80
E_PAD = 327680
EC = E_PAD // NW // K
B_PAD = 10240
BW = B_PAD // NW
BQ = BW // K

_mesh = plsc.VectorSubcoreMesh(core_axis_name="c", subcore_axis_name="s")


@functools.partial(
    pl.kernel,
    out_type=jax.ShapeDtypeStruct((NC, N_PAD, D), jnp.float32),
    mesh=_mesh,
    scratch_types=[
        pltpu.VMEM((EC // 2, K), jnp.int32),
        pltpu.VMEM((EC // 2, K), jnp.int32),
        pltpu.VMEM((K, D), jnp.float32),
        pltpu.VMEM((K, D), jnp.float32),
        pltpu.VMEM_SHARED((N_PAD, D), jnp.float32),
        pltpu.SemaphoreType.DMA,
    ],
)
def _edge_agg(emb_hbm, src_hbm, dst_hbm, acc_out,
              src_v, dst_v, rows_v, rows1_v, acc_sh, sem0):
    c = lax.axis_index("c")
    s = lax.axis_index("s")
    zero16 = jnp.zeros((16,), jnp.float32)

    def _zero_rows(i, _):
        for q in range(D // 16):
            rows_v[i, pl.ds(q * 16, 16)] = zero16
        return 0

    lax.fori_loop(0, K, _zero_rows, 0)
    my_base = s * ROWS_PER_TILE
    for q in range(ROWS_PER_TILE // K):
        pltpu.sync_copy(rows_v, acc_sh.at[pl.ds(my_base + q * K, K)])
    plsc.subcore_barrier()

    CB = EC // 2
    for h in range(2):
        pltpu.sync_copy(src_hbm.at[c, s, pl.ds(h * CB, CB)], src_v)
        pltpu.sync_copy(dst_hbm.at[c, s, pl.ds(h * CB, CB)], dst_v)
        pltpu.async_copy(emb_hbm.at[src_v.at[0]], rows_v, sem0)
        last = CB - 1

        def _pair(jj, _):
            j = 2 * jj
            pltpu.make_async_copy(emb_hbm.at[src_v.at[0]], rows_v, sem0).wait()
            pltpu.async_copy(emb_hbm.at[src_v.at[j + 1]], rows1_v, sem0)
            pltpu.sync_copy(rows_v, acc_sh.at[dst_v.at[j]], add=True)
            pltpu.make_async_copy(emb_hbm.at[src_v.at[0]], rows1_v, sem0).wait()
            pltpu.async_copy(emb_hbm.at[src_v.at[jnp.minimum(j + 2, last)]],
                             rows_v, sem0)
            pltpu.sync_copy(rows1_v, acc_sh.at[dst_v.at[j + 1]], add=True)
            return 0

        lax.fori_loop(0, CB // 2, _pair, 0)
        pltpu.make_async_copy(emb_hbm.at[src_v.at[0]], rows_v, sem0).wait()
    plsc.subcore_barrier()

    pltpu.sync_copy(acc_sh.at[pl.ds(my_base, ROWS_PER_TILE)],
                    acc_out.at[c, pl.ds(my_base, ROWS_PER_TILE)])


@functools.partial(
    pl.kernel,
    out_type=jax.ShapeDtypeStruct((NC, N_PAD, D), jnp.float32),
    mesh=_mesh,
    scratch_types=[
        pltpu.VMEM((EC, K), jnp.int32),
        pltpu.VMEM((K, D), jnp.float32),
        pltpu.VMEM_SHARED((N_PAD, D), jnp.float32),
    ],
)
def _degrees(dst_hbm, deg_out, dst_v, ones_v, deg_sh):
    c = lax.axis_index("c")
    s = lax.axis_index("s")
    zero16 = jnp.zeros((16,), jnp.float32)

    def _fill(val):
        def _f(i, _):
            for q in range(D // 16):
                ones_v[i, pl.ds(q * 16, 16)] = zero16 + val
            return 0
        lax.fori_loop(0, K, _f, 0)

    _fill(0.0)
    my_base = s * ROWS_PER_TILE
    for q in range(ROWS_PER_TILE // K):
        pltpu.sync_copy(ones_v, deg_sh.at[pl.ds(my_base + q * K, K)])
    _fill(1.0)
    plsc.subcore_barrier()

    pltpu.sync_copy(dst_hbm.at[c, s], dst_v)

    def _chunk(j, _):
        pltpu.sync_copy(ones_v, deg_sh.at[dst_v.at[j]], add=True)
        return 0

    lax.fori_loop(0, EC, _chunk, 0)
    plsc.subcore_barrier()

    pltpu.sync_copy(deg_sh.at[pl.ds(my_base, ROWS_PER_TILE)],
                    deg_out.at[c, pl.ds(my_base, ROWS_PER_TILE)])


def _linear_body(emb_ref, acc_ref, deg_ref, wt_ref, out_ref):
    a = acc_ref[0] + acc_ref[1]
    dsum = deg_ref[0] + deg_ref[1]
    deg = jnp.max(dsum, axis=1, keepdims=True)
    r = 1.0 / jnp.maximum(deg, 1.0)
    x1 = emb_ref[...]
    x2 = a * r
    w1 = wt_ref[:D, :]
    w2 = wt_ref[D:, :]
    t = (jnp.dot(x1, w1, preferred_element_type=jnp.float32,
                 precision=lax.Precision.HIGHEST)
         + jnp.dot(x2, w2, preferred_element_type=jnp.float32,
                   precision=lax.Precision.HIGHEST))
    out_ref[...] = jnp.maximum(t, 0.0)


@functools.partial(
    pl.kernel,
    out_type=jax.ShapeDtypeStruct((B_PAD, D), jnp.float32),
    mesh=_mesh,
    scratch_types=[
        pltpu.VMEM((BQ, K), jnp.int32),
        pltpu.VMEM((BW, D), jnp.float32),
    ],
)
def _batch_gather(t_hbm, nid_hbm, out_hbm, idx_v, rows_v):
    c = lax.axis_index("c")
    s = lax.axis_index("s")
    wid = c * NS + s
    pltpu.sync_copy(nid_hbm.at[c, s], idx_v)
    for q in range(BQ):
        pltpu.sync_copy(t_hbm.at[idx_v.at[q]], rows_v.at[pl.ds(q * K, K)])
    pltpu.sync_copy(rows_v, out_hbm.at[pl.ds(wid * BW, BW)])


def kernel(node_ids, edge_index, emb_table, W):
    npad = E_PAD - N_EDGES
    ar = jnp.arange(npad, dtype=edge_index.dtype)
    src = jnp.concatenate(
        [edge_index[0], ar % N_NODES]).reshape(NC, NS, EC, K)
    dst = jnp.concatenate(
        [edge_index[1], N_NODES + ar % (N_PAD - N_NODES)]
    ).reshape(NC, NS, EC, K)
    acc = _edge_agg(emb_table, src, dst)
    deg = _degrees(dst)

    R = 1000
    t = pl.pallas_call(
        _linear_body,
        grid=(N_NODES // R,),
        in_specs=[
            pl.BlockSpec((R, D), lambda i: (i, 0)),
            pl.BlockSpec((NC, R, D), lambda i: (0, i, 0)),
            pl.BlockSpec((NC, R, D), lambda i: (0, i, 0)),
            pl.BlockSpec((2 * D, D), lambda i: (0, 0)),
        ],
        out_specs=pl.BlockSpec((R, D), lambda i: (i, 0)),
        out_shape=jax.ShapeDtypeStruct((N_NODES, D), jnp.float32),
    )(emb_table, acc, deg, W.T)

    nid = jnp.concatenate(
        [node_ids, jnp.zeros((B_PAD - N_NODES,), node_ids.dtype)]
    ).reshape(NC, NS, BQ, K)
    out = _batch_gather(t, nid)
    return out[:N_NODES]
